# baseline (speedup 1.0000x reference)
import jax
import jax.numpy as jnp
from jax import lax
from jax.experimental import pallas as pl
from jax.experimental.pallas import tpu as pltpu
from jax.experimental.pallas import tpu_sc as plsc

NC, NS, L = 2, 16, 16
NW = NC * NS
G = 64

_MESH = dict(core_axis_name="c", subcore_axis_name="s", num_cores=NC,
             num_subcores=NS)
_SC_PARAMS = pltpu.CompilerParams(needs_layout_passes=False)


def _wid():
    return lax.axis_index("s") * NC + lax.axis_index("c")


def _zero(ref, n):
    @plsc.parallel_loop(0, n // L, unroll=8)
    def body(i):
        ref[pl.ds(i * L, L)] = jnp.zeros((L,), jnp.float32)


def _slw(n):
    s = -(-n // NS)
    s = -(-s // 16) * 16
    return s, NS * s


def _sc_deg(dst, ew, n):
    ep = dst.shape[0]
    c = ep // NW
    slw, np_ = _slw(n)

    def body(dst_hbm, ew_hbm, out_hbm, d_v, w_v, acc_v, sem1, sem2):
        wid = _wid()
        base = wid * c
        cp1 = pltpu.async_copy(dst_hbm.at[pl.ds(base, c)], d_v, sem1)
        cp2 = pltpu.async_copy(ew_hbm.at[pl.ds(base, c)], w_v, sem2)
        _zero(acc_v, np_)
        cp1.wait()
        cp2.wait()

        @plsc.parallel_loop(0, c // L, unroll=8)
        def step(i):
            sl = pl.ds(i * L, L)
            plsc.addupdate_scatter(acc_v, [d_v[sl]], w_v[sl])
        pltpu.sync_copy(acc_v, out_hbm.at[pl.ds(wid * np_, np_)])

    out = pl.kernel(
        body,
        out_type=jax.ShapeDtypeStruct((NW * np_,), jnp.float32),
        mesh=plsc.VectorSubcoreMesh(**_MESH),
        compiler_params=_SC_PARAMS,
        scratch_types=[
            pltpu.VMEM((c,), jnp.int32),
            pltpu.VMEM((c,), jnp.float32),
            pltpu.VMEM((np_,), jnp.float32),
            pltpu.SemaphoreType.DMA,
            pltpu.SemaphoreType.DMA,
        ],
    )(dst, ew)
    return out


def _sc_agg(src, dst, ew, degp, xf):
    ep = src.shape[0]
    c = ep // NW
    n = xf.shape[0] // 5
    slw, np_ = _slw(n)

    def body(src_hbm, dst_hbm, ew_hbm, degp_hbm, xf_hbm, norm_hbm, agg_hbm,
             dis_hbm, s_v, d_v, w_v, nrm_v, dis_v, tab0_v, tab1_v, acc0_v,
             acc1_v, acc2_v, acc3_v, acc5_v, big_v, dtmp_v, dis_sh,
             sem_in, sem_t0, sem_t1, sem_t2, sem_t3, sem_a, sem_a5, sem_dp):
        wid = _wid()
        sid = lax.axis_index("s")
        base = wid * c

        cps_dp = [pltpu.async_copy(
            degp_hbm.at[pl.ds(w * np_ + sid * slw, slw)],
            big_v.at[pl.ds(w * slw, slw)], sem_dp) for w in range(NW)]
        cp_s = pltpu.async_copy(src_hbm.at[pl.ds(base, c)], s_v, sem_in)
        cp_d = pltpu.async_copy(dst_hbm.at[pl.ds(base, c)], d_v, sem_in)
        cp_w = pltpu.async_copy(ew_hbm.at[pl.ds(base, c)], w_v, sem_in)
        cp_t0 = pltpu.async_copy(xf_hbm.at[pl.ds(0, n)], tab0_v, sem_t0)
        cp_t1 = pltpu.async_copy(xf_hbm.at[pl.ds(n, n)], tab1_v, sem_t1)
        _zero(acc0_v, n)
        _zero(acc1_v, n)
        _zero(acc2_v, n)
        _zero(acc3_v, n)
        _zero(acc5_v, n)
        for cp in cps_dp:
            cp.wait()

        @plsc.parallel_loop(0, slw // L, unroll=4)
        def dstep(j):
            s16 = big_v[pl.ds(j * L, L)]
            for w in range(1, NW):
                s16 = s16 + big_v[pl.ds(w * slw + j * L, L)]
            deg = s16 + 1.0
            ti = plsc.bitcast(deg, jnp.int32)
            y = plsc.bitcast(0x5F3759DF - (ti >> 1), jnp.float32)
            for _ in range(3):
                y = y * (1.5 - 0.5 * deg * y * y)
            dtmp_v[pl.ds(j * L, L)] = y
        pltpu.sync_copy(dtmp_v, dis_sh.at[pl.ds(sid * slw, slw)])

        @pl.when(lax.axis_index("c") == 0)
        def _():
            pltpu.sync_copy(dtmp_v, dis_hbm.at[pl.ds(sid * slw, slw)])

        tab2 = big_v.at[pl.ds(0, n)]
        tab3 = big_v.at[pl.ds((NW * slw) // 2, n)]
        cp_t2 = pltpu.async_copy(xf_hbm.at[pl.ds(2 * n, n)], tab2, sem_t2)
        cp_t3 = pltpu.async_copy(xf_hbm.at[pl.ds(3 * n, n)], tab3, sem_t3)
        plsc.subcore_barrier()
        pltpu.sync_copy(dis_sh, dis_v)
        cp_s.wait()
        cp_d.wait()
        cp_w.wait()
        cp_t0.wait()
        cp_t1.wait()
        cp_t2.wait()
        cp_t3.wait()

        @plsc.parallel_loop(0, c // L, unroll=4)
        def stepB(i):
            sl = pl.ds(i * L, L)
            si = s_v[sl]
            di = d_v[sl]
            dd_s = plsc.load_gather(dis_v, [si])
            dd_d = plsc.load_gather(dis_v, [di])
            nm = dd_s * w_v[sl] * dd_d
            nrm_v[sl] = nm
            plsc.addupdate_scatter(acc5_v, [di], nm)
            plsc.addupdate_scatter(
                acc0_v, [di], plsc.load_gather(tab0_v, [si]) * nm)
            plsc.addupdate_scatter(
                acc1_v, [di], plsc.load_gather(tab1_v, [si]) * nm)
            plsc.addupdate_scatter(
                acc2_v, [di], plsc.load_gather(tab2, [si]) * nm)
            plsc.addupdate_scatter(
                acc3_v, [di], plsc.load_gather(tab3, [si]) * nm)
        cp_n = pltpu.async_copy(nrm_v, norm_hbm.at[pl.ds(base, c)], sem_in)
        cp_a5 = pltpu.async_copy(
            acc5_v, agg_hbm.at[pl.ds(wid * 6 * n + 5 * n, n)], sem_a5)
        wb = [pltpu.async_copy(
            a, agg_hbm.at[pl.ds(wid * 6 * n + f * n, n)], sem_a)
            for f, a in enumerate((acc0_v, acc1_v, acc2_v, acc3_v))]

        cp_a5.wait()
        cp_t4 = pltpu.async_copy(xf_hbm.at[pl.ds(4 * n, n)], tab0_v, sem_t0)
        _zero(acc5_v, n)
        cp_t4.wait()

        @plsc.parallel_loop(0, c // L, unroll=8)
        def stepC(i):
            sl = pl.ds(i * L, L)
            plsc.addupdate_scatter(
                acc5_v, [d_v[sl]],
                plsc.load_gather(tab0_v, [s_v[sl]]) * nrm_v[sl])
        pltpu.sync_copy(acc5_v, agg_hbm.at[pl.ds(wid * 6 * n + 4 * n, n)])
        for cp in wb:
            cp.wait()
        cp_n.wait()

    normE, aggp, dis = pl.kernel(
        body,
        out_type=(jax.ShapeDtypeStruct((ep,), jnp.float32),
                  jax.ShapeDtypeStruct((NW * 6 * n,), jnp.float32),
                  jax.ShapeDtypeStruct((np_,), jnp.float32)),
        mesh=plsc.VectorSubcoreMesh(**_MESH),
        compiler_params=_SC_PARAMS,
        scratch_types=[
            pltpu.VMEM((c,), jnp.int32),
            pltpu.VMEM((c,), jnp.int32),
            pltpu.VMEM((c,), jnp.float32),
            pltpu.VMEM((c,), jnp.float32),
            pltpu.VMEM((np_,), jnp.float32),
            pltpu.VMEM((n,), jnp.float32),
            pltpu.VMEM((n,), jnp.float32),
            pltpu.VMEM((n,), jnp.float32),
            pltpu.VMEM((n,), jnp.float32),
            pltpu.VMEM((n,), jnp.float32),
            pltpu.VMEM((n,), jnp.float32),
            pltpu.VMEM((n,), jnp.float32),
            pltpu.VMEM((NW * slw,), jnp.float32),
            pltpu.VMEM((slw,), jnp.float32),
            pltpu.VMEM_SHARED((np_,), jnp.float32),
            pltpu.SemaphoreType.DMA,
            pltpu.SemaphoreType.DMA,
            pltpu.SemaphoreType.DMA,
            pltpu.SemaphoreType.DMA,
            pltpu.SemaphoreType.DMA,
            pltpu.SemaphoreType.DMA,
            pltpu.SemaphoreType.DMA,
            pltpu.SemaphoreType.DMA,
        ],
    )(src, dst, ew, degp, xf)
    return normE, aggp.reshape(NW, 6, n), dis[:n]


def _tc_dense(aggp, dis, xt, W0, b0, W1, b1, W2):
    n = aggp.shape[2]

    def body(aggp_ref, dis_ref, xt_ref, W0_ref, b0_ref, W1_ref, b1_ref,
             W2_ref, z_ref):
        agg = jnp.sum(aggp_ref[...], axis=0)
        dis = dis_ref[...]
        dis2 = dis * dis
        axT = agg[:5] + xt_ref[...] * dis2[None, :]
        a1 = agg[5] + dis2
        W01T = lax.dot_general(
            W1_ref[...], W0_ref[...], (((0,), (1,)), ((), ())),
            preferred_element_type=jnp.float32)
        bW1 = jnp.dot(b0_ref[...], W1_ref[...],
                      preferred_element_type=jnp.float32)
        h = lax.dot_general(W01T, axT, (((1,), (0,)), ((), ())),
                            preferred_element_type=jnp.float32)
        h = h + bW1[:, None] * a1[None, :] + b1_ref[...][:, None]
        h = jnp.maximum(h, 0.0)
        zT = lax.dot_general(W2_ref[...], h, (((0,), (0,)), ((), ())),
                             preferred_element_type=jnp.float32)
        z_ref[...] = zT[0]

    return pl.pallas_call(
        body, out_shape=jax.ShapeDtypeStruct((n,), jnp.float32))(
            aggp, dis, xt, W0, b0, W1, b1, W2)


def _sc_agg2(src, dst, normE, z):
    ep = src.shape[0]
    c = ep // NW
    n = z.shape[0]

    def body(src_hbm, dst_hbm, nrm_hbm, z_hbm, out_hbm,
             s_v, d_v, nrm_v, z_v, acc_v, sem_in):
        wid = _wid()
        base = wid * c
        cp_s = pltpu.async_copy(src_hbm.at[pl.ds(base, c)], s_v, sem_in)
        cp_d = pltpu.async_copy(dst_hbm.at[pl.ds(base, c)], d_v, sem_in)
        cp_n = pltpu.async_copy(nrm_hbm.at[pl.ds(base, c)], nrm_v, sem_in)
        cp_z = pltpu.async_copy(z_hbm, z_v, sem_in)
        _zero(acc_v, n)
        cp_s.wait()
        cp_d.wait()
        cp_n.wait()
        cp_z.wait()

        @plsc.parallel_loop(0, c // L, unroll=8)
        def step(i):
            sl = pl.ds(i * L, L)
            zs = plsc.load_gather(z_v, [s_v[sl]])
            plsc.addupdate_scatter(acc_v, [d_v[sl]], zs * nrm_v[sl])
        pltpu.sync_copy(acc_v, out_hbm.at[pl.ds(wid * n, n)])

    out = pl.kernel(
        body,
        out_type=jax.ShapeDtypeStruct((NW * n,), jnp.float32),
        mesh=plsc.VectorSubcoreMesh(**_MESH),
        compiler_params=_SC_PARAMS,
        scratch_types=[
            pltpu.VMEM((c,), jnp.int32),
            pltpu.VMEM((c,), jnp.int32),
            pltpu.VMEM((c,), jnp.float32),
            pltpu.VMEM((n,), jnp.float32),
            pltpu.VMEM((n,), jnp.float32),
            pltpu.SemaphoreType.DMA,
        ],
    )(src, dst, normE, z)
    return out.reshape(NW, n)


def _tc_softmax(out2p, dis, z, b2, batch):
    n = out2p.shape[1]
    b2r = jnp.reshape(b2, (1, 1))

    def body(op_ref, dis_ref, z_ref, b2_ref, batch_ref, out_ref):
        dis = dis_ref[...]
        o = (jnp.sum(op_ref[...], axis=0) + dis * dis * z_ref[...]
             + b2_ref[0, 0])
        seg = batch_ref[...]
        gids = lax.broadcasted_iota(jnp.int32, (G, n), 0)
        onehot = gids == seg[None, :]
        m = jnp.max(jnp.where(onehot, o[None, :], -jnp.inf), axis=1)
        mb = jnp.sum(jnp.where(onehot, m[:, None], 0.0), axis=0)
        e = jnp.exp(o - mb)
        den = jnp.sum(jnp.where(onehot, e[None, :], 0.0), axis=1)
        denb = jnp.sum(jnp.where(onehot, den[:, None], 0.0), axis=0)
        out_ref[...] = e / denb

    return pl.pallas_call(
        body, out_shape=jax.ShapeDtypeStruct((n,), jnp.float32))(
            out2p, dis, z, b2r, batch)


def kernel(x, edge_index, edge_attr, batch, W0, b0, W1, b1, W2, b2):
    n = x.shape[0]
    e = edge_attr.shape[0]
    src, dst = edge_index[0], edge_index[1]

    ep = -(-e // (NW * L)) * (NW * L)
    if ep != e:
        pad = ep - e
        src = jnp.concatenate([src, jnp.zeros((pad,), src.dtype)])
        dst = jnp.concatenate([dst, jnp.zeros((pad,), dst.dtype)])
        ew = jnp.concatenate([edge_attr, jnp.zeros((pad,), edge_attr.dtype)])
    else:
        ew = edge_attr
    xt = x.T
    xf = jnp.reshape(xt, (-1,))

    degp = _sc_deg(dst, ew, n)
    normE, aggp, dis = _sc_agg(src, dst, ew, degp, xf)
    z = _tc_dense(aggp, dis, xt, W0, b0, W1, b1, W2)
    out2p = _sc_agg2(src, dst, normE, z)
    return _tc_softmax(out2p, dis, z, b2, batch)


# speedup vs baseline: 1.0052x; 1.0052x over previous
import jax
import jax.numpy as jnp
from jax import lax
from jax.experimental import pallas as pl
from jax.experimental.pallas import tpu as pltpu
from jax.experimental.pallas import tpu_sc as plsc

NC, NS, L = 2, 16, 16
NW = NC * NS
G = 64

_MESH = dict(core_axis_name="c", subcore_axis_name="s", num_cores=NC,
             num_subcores=NS)
_SC_PARAMS = pltpu.CompilerParams(needs_layout_passes=False)


def _wid():
    return lax.axis_index("s") * NC + lax.axis_index("c")


def _zero(ref, n):
    @plsc.parallel_loop(0, n // L, unroll=8)
    def body(i):
        ref[pl.ds(i * L, L)] = jnp.zeros((L,), jnp.float32)


def _slw(n):
    s = -(-n // NS)
    s = -(-s // 16) * 16
    return s, NS * s


def _sc_deg(dst, ew, n):
    ep = dst.shape[0]
    c = ep // NW
    slw, np_ = _slw(n)

    def body(dst_hbm, ew_hbm, out_hbm, d_v, w_v, acc_v, sem1, sem2):
        wid = _wid()
        base = wid * c
        cp1 = pltpu.async_copy(dst_hbm.at[pl.ds(base, c)], d_v, sem1)
        cp2 = pltpu.async_copy(ew_hbm.at[pl.ds(base, c)], w_v, sem2)
        _zero(acc_v, np_)
        cp1.wait()
        cp2.wait()

        @plsc.parallel_loop(0, c // L, unroll=8)
        def step(i):
            sl = pl.ds(i * L, L)
            plsc.addupdate_scatter(acc_v, [d_v[sl]], w_v[sl])
        pltpu.sync_copy(acc_v, out_hbm.at[pl.ds(wid * np_, np_)])

    out = pl.kernel(
        body,
        out_type=jax.ShapeDtypeStruct((NW * np_,), jnp.float32),
        mesh=plsc.VectorSubcoreMesh(**_MESH),
        compiler_params=_SC_PARAMS,
        scratch_types=[
            pltpu.VMEM((c,), jnp.int32),
            pltpu.VMEM((c,), jnp.float32),
            pltpu.VMEM((np_,), jnp.float32),
            pltpu.SemaphoreType.DMA,
            pltpu.SemaphoreType.DMA,
        ],
    )(dst, ew)
    return out


def _sc_agg(src, dst, ew, degp, xf):
    ep = src.shape[0]
    c = ep // NW
    n = xf.shape[0] // 5
    slw, np_ = _slw(n)

    def body(src_hbm, dst_hbm, ew_hbm, degp_hbm, xf_hbm, norm_hbm, agg_hbm,
             dis_hbm, s_v, d_v, w_v, nrm_v, dis_v, tab0_v, tab1_v, acc0_v,
             acc1_v, acc2_v, acc3_v, acc5_v, big_v, dtmp_v, dis_sh,
             sem_in, sem_t0, sem_t1, sem_t2, sem_t3, sem_a, sem_a5, sem_dp):
        wid = _wid()
        sid = lax.axis_index("s")
        base = wid * c

        cps_dp = [pltpu.async_copy(
            degp_hbm.at[pl.ds(w * np_ + sid * slw, slw)],
            big_v.at[pl.ds(w * slw, slw)], sem_dp) for w in range(NW)]
        cp_s = pltpu.async_copy(src_hbm.at[pl.ds(base, c)], s_v, sem_in)
        cp_d = pltpu.async_copy(dst_hbm.at[pl.ds(base, c)], d_v, sem_in)
        cp_w = pltpu.async_copy(ew_hbm.at[pl.ds(base, c)], w_v, sem_in)
        cp_t0 = pltpu.async_copy(xf_hbm.at[pl.ds(0, n)], tab0_v, sem_t0)
        cp_t1 = pltpu.async_copy(xf_hbm.at[pl.ds(n, n)], tab1_v, sem_t1)
        _zero(acc0_v, n)
        _zero(acc1_v, n)
        _zero(acc2_v, n)
        _zero(acc3_v, n)
        _zero(acc5_v, n)
        for cp in cps_dp:
            cp.wait()

        @plsc.parallel_loop(0, slw // L, unroll=4)
        def dstep(j):
            s16 = big_v[pl.ds(j * L, L)]
            for w in range(1, NW):
                s16 = s16 + big_v[pl.ds(w * slw + j * L, L)]
            deg = s16 + 1.0
            ti = plsc.bitcast(deg, jnp.int32)
            y = plsc.bitcast(0x5F3759DF - (ti >> 1), jnp.float32)
            for _ in range(3):
                y = y * (1.5 - 0.5 * deg * y * y)
            dtmp_v[pl.ds(j * L, L)] = y
        pltpu.sync_copy(dtmp_v, dis_sh.at[pl.ds(sid * slw, slw)])

        @pl.when(lax.axis_index("c") == 0)
        def _():
            pltpu.sync_copy(dtmp_v, dis_hbm.at[pl.ds(sid * slw, slw)])

        tab2 = big_v.at[pl.ds(0, n)]
        tab3 = big_v.at[pl.ds((NW * slw) // 2, n)]
        cp_t2 = pltpu.async_copy(xf_hbm.at[pl.ds(2 * n, n)], tab2, sem_t2)
        cp_t3 = pltpu.async_copy(xf_hbm.at[pl.ds(3 * n, n)], tab3, sem_t3)
        plsc.subcore_barrier()
        pltpu.sync_copy(dis_sh, dis_v)
        cp_s.wait()
        cp_d.wait()
        cp_w.wait()

        @plsc.parallel_loop(0, c // L, unroll=8)
        def normstep(i):
            sl = pl.ds(i * L, L)
            dd_s = plsc.load_gather(dis_v, [s_v[sl]])
            dd_d = plsc.load_gather(dis_v, [d_v[sl]])
            nm = dd_s * w_v[sl] * dd_d
            nrm_v[sl] = nm
            plsc.addupdate_scatter(acc5_v, [d_v[sl]], nm)
        cp_n = pltpu.async_copy(nrm_v, norm_hbm.at[pl.ds(base, c)], sem_in)
        cp_a5 = pltpu.async_copy(
            acc5_v, agg_hbm.at[pl.ds(wid * 6 * n + 5 * n, n)], sem_a5)
        cp_t0.wait()
        cp_t1.wait()
        cp_t2.wait()
        cp_t3.wait()

        @plsc.parallel_loop(0, c // L, unroll=4)
        def stepB(i):
            sl = pl.ds(i * L, L)
            si = s_v[sl]
            di = d_v[sl]
            nm = nrm_v[sl]
            plsc.addupdate_scatter(
                acc0_v, [di], plsc.load_gather(tab0_v, [si]) * nm)
            plsc.addupdate_scatter(
                acc1_v, [di], plsc.load_gather(tab1_v, [si]) * nm)
            plsc.addupdate_scatter(
                acc2_v, [di], plsc.load_gather(tab2, [si]) * nm)
            plsc.addupdate_scatter(
                acc3_v, [di], plsc.load_gather(tab3, [si]) * nm)
        wb = [pltpu.async_copy(
            a, agg_hbm.at[pl.ds(wid * 6 * n + f * n, n)], sem_a)
            for f, a in enumerate((acc0_v, acc1_v, acc2_v, acc3_v))]

        cp_a5.wait()
        cp_t4 = pltpu.async_copy(xf_hbm.at[pl.ds(4 * n, n)], tab0_v, sem_t0)
        _zero(acc5_v, n)
        cp_t4.wait()

        @plsc.parallel_loop(0, c // L, unroll=8)
        def stepC(i):
            sl = pl.ds(i * L, L)
            plsc.addupdate_scatter(
                acc5_v, [d_v[sl]],
                plsc.load_gather(tab0_v, [s_v[sl]]) * nrm_v[sl])
        pltpu.sync_copy(acc5_v, agg_hbm.at[pl.ds(wid * 6 * n + 4 * n, n)])
        for cp in wb:
            cp.wait()
        cp_n.wait()

    normE, aggp, dis = pl.kernel(
        body,
        out_type=(jax.ShapeDtypeStruct((ep,), jnp.float32),
                  jax.ShapeDtypeStruct((NW * 6 * n,), jnp.float32),
                  jax.ShapeDtypeStruct((np_,), jnp.float32)),
        mesh=plsc.VectorSubcoreMesh(**_MESH),
        compiler_params=_SC_PARAMS,
        scratch_types=[
            pltpu.VMEM((c,), jnp.int32),
            pltpu.VMEM((c,), jnp.int32),
            pltpu.VMEM((c,), jnp.float32),
            pltpu.VMEM((c,), jnp.float32),
            pltpu.VMEM((np_,), jnp.float32),
            pltpu.VMEM((n,), jnp.float32),
            pltpu.VMEM((n,), jnp.float32),
            pltpu.VMEM((n,), jnp.float32),
            pltpu.VMEM((n,), jnp.float32),
            pltpu.VMEM((n,), jnp.float32),
            pltpu.VMEM((n,), jnp.float32),
            pltpu.VMEM((n,), jnp.float32),
            pltpu.VMEM((NW * slw,), jnp.float32),
            pltpu.VMEM((slw,), jnp.float32),
            pltpu.VMEM_SHARED((np_,), jnp.float32),
            pltpu.SemaphoreType.DMA,
            pltpu.SemaphoreType.DMA,
            pltpu.SemaphoreType.DMA,
            pltpu.SemaphoreType.DMA,
            pltpu.SemaphoreType.DMA,
            pltpu.SemaphoreType.DMA,
            pltpu.SemaphoreType.DMA,
            pltpu.SemaphoreType.DMA,
        ],
    )(src, dst, ew, degp, xf)
    return normE, aggp.reshape(NW, 6, n), dis[:n]


def _tc_dense(aggp, dis, xt, W0, b0, W1, b1, W2):
    n = aggp.shape[2]

    def body(aggp_ref, dis_ref, xt_ref, W0_ref, b0_ref, W1_ref, b1_ref,
             W2_ref, z_ref):
        agg = jnp.sum(aggp_ref[...], axis=0)
        dis = dis_ref[...]
        dis2 = dis * dis
        axT = agg[:5] + xt_ref[...] * dis2[None, :]
        a1 = agg[5] + dis2
        W01T = lax.dot_general(
            W1_ref[...], W0_ref[...], (((0,), (1,)), ((), ())),
            preferred_element_type=jnp.float32)
        bW1 = jnp.dot(b0_ref[...], W1_ref[...],
                      preferred_element_type=jnp.float32)
        h = lax.dot_general(W01T, axT, (((1,), (0,)), ((), ())),
                            preferred_element_type=jnp.float32)
        h = h + bW1[:, None] * a1[None, :] + b1_ref[...][:, None]
        h = jnp.maximum(h, 0.0)
        zT = lax.dot_general(W2_ref[...], h, (((0,), (0,)), ((), ())),
                             preferred_element_type=jnp.float32)
        z_ref[...] = zT[0]

    return pl.pallas_call(
        body, out_shape=jax.ShapeDtypeStruct((n,), jnp.float32))(
            aggp, dis, xt, W0, b0, W1, b1, W2)


def _sc_agg2(src, dst, normE, z):
    ep = src.shape[0]
    c = ep // NW
    n = z.shape[0]

    def body(src_hbm, dst_hbm, nrm_hbm, z_hbm, out_hbm,
             s_v, d_v, nrm_v, z_v, acc_v, sem_in):
        wid = _wid()
        base = wid * c
        cp_s = pltpu.async_copy(src_hbm.at[pl.ds(base, c)], s_v, sem_in)
        cp_d = pltpu.async_copy(dst_hbm.at[pl.ds(base, c)], d_v, sem_in)
        cp_n = pltpu.async_copy(nrm_hbm.at[pl.ds(base, c)], nrm_v, sem_in)
        cp_z = pltpu.async_copy(z_hbm, z_v, sem_in)
        _zero(acc_v, n)
        cp_s.wait()
        cp_d.wait()
        cp_n.wait()
        cp_z.wait()

        @plsc.parallel_loop(0, c // L, unroll=8)
        def step(i):
            sl = pl.ds(i * L, L)
            zs = plsc.load_gather(z_v, [s_v[sl]])
            plsc.addupdate_scatter(acc_v, [d_v[sl]], zs * nrm_v[sl])
        pltpu.sync_copy(acc_v, out_hbm.at[pl.ds(wid * n, n)])

    out = pl.kernel(
        body,
        out_type=jax.ShapeDtypeStruct((NW * n,), jnp.float32),
        mesh=plsc.VectorSubcoreMesh(**_MESH),
        compiler_params=_SC_PARAMS,
        scratch_types=[
            pltpu.VMEM((c,), jnp.int32),
            pltpu.VMEM((c,), jnp.int32),
            pltpu.VMEM((c,), jnp.float32),
            pltpu.VMEM((n,), jnp.float32),
            pltpu.VMEM((n,), jnp.float32),
            pltpu.SemaphoreType.DMA,
        ],
    )(src, dst, normE, z)
    return out.reshape(NW, n)


def _tc_softmax(out2p, dis, z, b2, batch):
    n = out2p.shape[1]
    b2r = jnp.reshape(b2, (1, 1))

    def body(op_ref, dis_ref, z_ref, b2_ref, batch_ref, out_ref):
        dis = dis_ref[...]
        o = (jnp.sum(op_ref[...], axis=0) + dis * dis * z_ref[...]
             + b2_ref[0, 0])
        seg = batch_ref[...]
        gids = lax.broadcasted_iota(jnp.int32, (G, n), 0)
        onehot = gids == seg[None, :]
        m = jnp.max(jnp.where(onehot, o[None, :], -jnp.inf), axis=1)
        mb = jnp.sum(jnp.where(onehot, m[:, None], 0.0), axis=0)
        e = jnp.exp(o - mb)
        den = jnp.sum(jnp.where(onehot, e[None, :], 0.0), axis=1)
        denb = jnp.sum(jnp.where(onehot, den[:, None], 0.0), axis=0)
        out_ref[...] = e / denb

    return pl.pallas_call(
        body, out_shape=jax.ShapeDtypeStruct((n,), jnp.float32))(
            out2p, dis, z, b2r, batch)


def kernel(x, edge_index, edge_attr, batch, W0, b0, W1, b1, W2, b2):
    n = x.shape[0]
    e = edge_attr.shape[0]
    src, dst = edge_index[0], edge_index[1]

    ep = -(-e // (NW * L)) * (NW * L)
    if ep != e:
        pad = ep - e
        src = jnp.concatenate([src, jnp.zeros((pad,), src.dtype)])
        dst = jnp.concatenate([dst, jnp.zeros((pad,), dst.dtype)])
        ew = jnp.concatenate([edge_attr, jnp.zeros((pad,), edge_attr.dtype)])
    else:
        ew = edge_attr
    xt = x.T
    xf = jnp.reshape(xt, (-1,))

    degp = _sc_deg(dst, ew, n)
    normE, aggp, dis = _sc_agg(src, dst, ew, degp, xf)
    z = _tc_dense(aggp, dis, xt, W0, b0, W1, b1, W2)
    out2p = _sc_agg2(src, dst, normE, z)
    return _tc_softmax(out2p, dis, z, b2, batch)
